# baseline (speedup 1.0000x reference)
import functools

import jax
import jax.numpy as jnp
from jax.experimental import pallas as pl
from jax.experimental.pallas import tpu as pltpu

_HIGH = jax.lax.Precision.HIGHEST


def _dot(a, b, dims):
    return jax.lax.dot_general(a, b, (dims, ((), ())), precision=_HIGH,
                               preferred_element_type=jnp.float32)


def _gnn_kernel(x_ref, mask_ref, w1_ref, b1_ref, wl1_ref, bl1_ref, wr1_ref,
                wl2_ref, bl2_ref, wr2_ref, out_ref):
    n = x_ref.shape[1]
    xb = x_ref[0]
    h = _dot(xb, w1_ref[...], (((1,), (1,)))) + b1_ref[...]

    g = _dot(h, h, (((1,), (1,))))
    row = jax.lax.broadcasted_iota(jnp.int32, (n, n), 0)
    col = jax.lax.broadcasted_iota(jnp.int32, (n, n), 1)
    diag = row == col
    nrm2 = jnp.sum(jnp.where(diag, g, 0.0), axis=1, keepdims=True)
    nrm = jnp.sqrt(nrm2)
    denom = jnp.maximum(nrm * nrm.reshape(1, n), 1e-8)
    a = jnp.where(diag, 0.0, g / denom)

    inv_cnt = 1.0 / (n - 1)
    agg1 = _dot(a, h, (((1,), (0,)))) * inv_cnt
    o1 = jnp.maximum(
        _dot(agg1, wl1_ref[...], (((1,), (1,))))
        + _dot(h, wr1_ref[...], (((1,), (1,))))
        + bl1_ref[...], 0.0)

    agg2 = _dot(a, o1, (((1,), (0,)))) * inv_cnt
    z = (_dot(wl2_ref[...], agg2, (((1,), (1,))))
         + _dot(wr2_ref[...], o1, (((1,), (1,))))
         + bl2_ref[...])
    out_ref[0] = jax.nn.sigmoid(z) * mask_ref[0]


@jax.jit
def kernel(x, mask_cls, W1, b1, Wl1, bl1, Wr1, Wl2, bl2, Wr2):
    B, N, H = x.shape
    full = lambda s: pl.BlockSpec(s, lambda i: (0,) * len(s))
    out = pl.pallas_call(
        _gnn_kernel,
        grid=(B,),
        in_specs=[
            pl.BlockSpec((1, N, H), lambda i: (i, 0, 0)),
            pl.BlockSpec((1, 1, N), lambda i: (i, 0, 0)),
            full(W1.shape),
            full((1, 128)),
            full(Wl1.shape),
            full((1, 64)),
            full(Wr1.shape),
            full(Wl2.shape),
            full((1, 1)),
            full(Wr2.shape),
        ],
        out_specs=pl.BlockSpec((1, 1, N), lambda i: (i, 0, 0)),
        out_shape=jax.ShapeDtypeStruct((B, 1, N), jnp.float32),
        compiler_params=pltpu.CompilerParams(
            dimension_semantics=("parallel",)),
    )(x, mask_cls.reshape(B, 1, N), W1, b1.reshape(1, 128), Wl1,
      bl1.reshape(1, 64), Wr1, Wl2, bl2.reshape(1, 1), Wr2)
    return out.reshape(B, N)


# speedup vs baseline: 1.9556x; 1.9556x over previous
import functools

import jax
import jax.numpy as jnp
from jax.experimental import pallas as pl
from jax.experimental.pallas import tpu as pltpu

_HIGH = jax.lax.Precision.DEFAULT


def _dot(a, b, dims):
    return jax.lax.dot_general(a, b, (dims, ((), ())), precision=_HIGH,
                               preferred_element_type=jnp.float32)


def _gnn_kernel(x_ref, mask_ref, w1_ref, b1_ref, wl1_ref, bl1_ref, wr1_ref,
                wl2_ref, bl2_ref, wr2_ref, out_ref):
    n = x_ref.shape[1]
    xb = x_ref[0]
    h = _dot(xb, w1_ref[...], (((1,), (1,)))) + b1_ref[...]

    g = _dot(h, h, (((1,), (1,))))
    row = jax.lax.broadcasted_iota(jnp.int32, (n, n), 0)
    col = jax.lax.broadcasted_iota(jnp.int32, (n, n), 1)
    diag = row == col
    nrm2 = jnp.sum(jnp.where(diag, g, 0.0), axis=1, keepdims=True)
    nrm = jnp.sqrt(nrm2)
    denom = jnp.maximum(nrm * nrm.reshape(1, n), 1e-8)
    a = jnp.where(diag, 0.0, g / denom)

    inv_cnt = 1.0 / (n - 1)
    agg1 = _dot(a, h, (((1,), (0,)))) * inv_cnt
    o1 = jnp.maximum(
        _dot(agg1, wl1_ref[...], (((1,), (1,))))
        + _dot(h, wr1_ref[...], (((1,), (1,))))
        + bl1_ref[...], 0.0)

    agg2 = _dot(a, o1, (((1,), (0,)))) * inv_cnt
    z = (_dot(wl2_ref[...], agg2, (((1,), (1,))))
         + _dot(wr2_ref[...], o1, (((1,), (1,))))
         + bl2_ref[...])
    out_ref[0] = jax.nn.sigmoid(z) * mask_ref[0]


@jax.jit
def kernel(x, mask_cls, W1, b1, Wl1, bl1, Wr1, Wl2, bl2, Wr2):
    B, N, H = x.shape
    full = lambda s: pl.BlockSpec(s, lambda i: (0,) * len(s))
    out = pl.pallas_call(
        _gnn_kernel,
        grid=(B,),
        in_specs=[
            pl.BlockSpec((1, N, H), lambda i: (i, 0, 0)),
            pl.BlockSpec((1, 1, N), lambda i: (i, 0, 0)),
            full(W1.shape),
            full((1, 128)),
            full(Wl1.shape),
            full((1, 64)),
            full(Wr1.shape),
            full(Wl2.shape),
            full((1, 1)),
            full(Wr2.shape),
        ],
        out_specs=pl.BlockSpec((1, 1, N), lambda i: (i, 0, 0)),
        out_shape=jax.ShapeDtypeStruct((B, 1, N), jnp.float32),
        compiler_params=pltpu.CompilerParams(
            dimension_semantics=("parallel",)),
    )(x, mask_cls.reshape(B, 1, N), W1, b1.reshape(1, 128), Wl1,
      bl1.reshape(1, 64), Wr1, Wl2, bl2.reshape(1, 1), Wr2)
    return out.reshape(B, N)


# speedup vs baseline: 2.0260x; 1.0360x over previous
import jax
import jax.numpy as jnp
from jax.experimental import pallas as pl
from jax.experimental.pallas import tpu as pltpu

_SUB = 2


def _dot(a, b, dims):
    return jax.lax.dot_general(a, b, (dims, ((), ())),
                               preferred_element_type=jnp.float32)


def _gnn_kernel(x_ref, mask_ref, w1_ref, b1_ref, wl1_ref, bl1_ref, wr1_ref,
                wl2_ref, bl2_ref, wr2_ref, out_ref):
    sub, n, hdim = x_ref.shape
    xb = x_ref[...].reshape(sub * n, hdim)
    h_all = _dot(xb, w1_ref[...], (((1,), (1,)))) + b1_ref[...]

    row = jax.lax.broadcasted_iota(jnp.int32, (n, n), 0)
    col = jax.lax.broadcasted_iota(jnp.int32, (n, n), 1)
    diag = row == col
    inv_cnt = 1.0 / (n - 1)

    for i in range(sub):
        h = h_all[i * n:(i + 1) * n]
        g = _dot(h, h, (((1,), (1,))))
        nrm2 = jnp.sum(jnp.where(diag, g, 0.0), axis=1, keepdims=True)
        nrm = jnp.sqrt(nrm2)
        denom = jnp.maximum(nrm * nrm.reshape(1, n), 1e-8)
        a = jnp.where(diag, 0.0, g / denom)

        agg1 = _dot(a, h, (((1,), (0,)))) * inv_cnt
        o1 = jnp.maximum(
            _dot(agg1, wl1_ref[...], (((1,), (1,))))
            + _dot(h, wr1_ref[...], (((1,), (1,))))
            + bl1_ref[...], 0.0)

        agg2 = _dot(a, o1, (((1,), (0,)))) * inv_cnt
        z = (_dot(wl2_ref[...], agg2, (((1,), (1,))))
             + _dot(wr2_ref[...], o1, (((1,), (1,))))
             + bl2_ref[...])
        out_ref[i] = jax.nn.sigmoid(z) * mask_ref[i]


@jax.jit
def kernel(x, mask_cls, W1, b1, Wl1, bl1, Wr1, Wl2, bl2, Wr2):
    B, N, H = x.shape
    full = lambda s: pl.BlockSpec(s, lambda i: (0,) * len(s))
    out = pl.pallas_call(
        _gnn_kernel,
        grid=(B // _SUB,),
        in_specs=[
            pl.BlockSpec((_SUB, N, H), lambda i: (i, 0, 0)),
            pl.BlockSpec((_SUB, 1, N), lambda i: (i, 0, 0)),
            full(W1.shape),
            full((1, 128)),
            full(Wl1.shape),
            full((1, 64)),
            full(Wr1.shape),
            full(Wl2.shape),
            full((1, 1)),
            full(Wr2.shape),
        ],
        out_specs=pl.BlockSpec((_SUB, 1, N), lambda i: (i, 0, 0)),
        out_shape=jax.ShapeDtypeStruct((B, 1, N), jnp.float32),
        compiler_params=pltpu.CompilerParams(
            dimension_semantics=("parallel",)),
    )(x, mask_cls.reshape(B, 1, N), W1, b1.reshape(1, 128), Wl1,
      bl1.reshape(1, 64), Wr1, Wl2, bl2.reshape(1, 1), Wr2)
    return out.reshape(B, N)


# speedup vs baseline: 2.0362x; 1.0050x over previous
import jax
import jax.numpy as jnp
from jax.experimental import pallas as pl
from jax.experimental.pallas import tpu as pltpu

_SUB = 4


def _dot(a, b, dims):
    return jax.lax.dot_general(a, b, (dims, ((), ())),
                               preferred_element_type=jnp.float32)


def _gnn_kernel(x_ref, mask_ref, w1_ref, b1_ref, wl1_ref, bl1_ref, wr1_ref,
                wl2_ref, bl2_ref, wr2_ref, out_ref):
    sub, n, hdim = x_ref.shape
    xb = x_ref[...].reshape(sub * n, hdim)
    h_all = _dot(xb, w1_ref[...], (((1,), (1,)))) + b1_ref[...]

    row = jax.lax.broadcasted_iota(jnp.int32, (n, n), 0)
    col = jax.lax.broadcasted_iota(jnp.int32, (n, n), 1)
    diag = row == col
    inv_cnt = 1.0 / (n - 1)

    for i in range(sub):
        h = h_all[i * n:(i + 1) * n]
        g = _dot(h, h, (((1,), (1,))))
        nrm2 = jnp.sum(jnp.where(diag, g, 0.0), axis=1, keepdims=True)
        nrm = jnp.sqrt(nrm2)
        denom = jnp.maximum(nrm * nrm.reshape(1, n), 1e-8)
        a = jnp.where(diag, 0.0, g / denom)

        agg1 = _dot(a, h, (((1,), (0,)))) * inv_cnt
        o1 = jnp.maximum(
            _dot(agg1, wl1_ref[...], (((1,), (1,))))
            + _dot(h, wr1_ref[...], (((1,), (1,))))
            + bl1_ref[...], 0.0)

        agg2 = _dot(a, o1, (((1,), (0,)))) * inv_cnt
        z = (_dot(wl2_ref[...], agg2, (((1,), (1,))))
             + _dot(wr2_ref[...], o1, (((1,), (1,))))
             + bl2_ref[...])
        out_ref[i] = jax.nn.sigmoid(z) * mask_ref[i]


@jax.jit
def kernel(x, mask_cls, W1, b1, Wl1, bl1, Wr1, Wl2, bl2, Wr2):
    B, N, H = x.shape
    full = lambda s: pl.BlockSpec(s, lambda i: (0,) * len(s))
    out = pl.pallas_call(
        _gnn_kernel,
        grid=(B // _SUB,),
        in_specs=[
            pl.BlockSpec((_SUB, N, H), lambda i: (i, 0, 0)),
            pl.BlockSpec((_SUB, 1, N), lambda i: (i, 0, 0)),
            full(W1.shape),
            full((1, 128)),
            full(Wl1.shape),
            full((1, 64)),
            full(Wr1.shape),
            full(Wl2.shape),
            full((1, 1)),
            full(Wr2.shape),
        ],
        out_specs=pl.BlockSpec((_SUB, 1, N), lambda i: (i, 0, 0)),
        out_shape=jax.ShapeDtypeStruct((B, 1, N), jnp.float32),
        compiler_params=pltpu.CompilerParams(
            dimension_semantics=("parallel",)),
    )(x, mask_cls.reshape(B, 1, N), W1, b1.reshape(1, 128), Wl1,
      bl1.reshape(1, 64), Wr1, Wl2, bl2.reshape(1, 1), Wr2)
    return out.reshape(B, N)
